# baseline (speedup 1.0000x reference)
import functools

import jax
import jax.numpy as jnp
from jax import lax
from jax.experimental import pallas as pl
from jax.experimental.pallas import tpu as pltpu
from jax.experimental.pallas import tpu_sc as plsc

N_USER = 10000
N_POI = 10000
EMBED = 32
BATCH = 4096
BM = 200
BM2 = 1000
BB = 512
PADW = 10112


def _prop1_body(a_ref, x_ref, u_ref, pt_ref, a16_ref):
    i = pl.program_id(0)
    a = a_ref[...]
    u = jnp.dot(a, x_ref[...], preferred_element_type=jnp.float32)
    u_ref[...] = u
    a16_ref[...] = jnp.pad(a.astype(jnp.bfloat16),
                           ((0, 0), (0, PADW - N_POI)))
    pt = jax.lax.dot_general(u, a, (((0,), (0,)), ((), ())),
                             preferred_element_type=jnp.float32)

    @pl.when(i == 0)
    def _init():
        pt_ref[...] = pt

    @pl.when(i != 0)
    def _acc():
        pt_ref[...] = pt_ref[...] + pt


def _propagate1(A, x):
    return pl.pallas_call(
        _prop1_body,
        grid=(N_USER // BM,),
        in_specs=[
            pl.BlockSpec((BM, N_POI), lambda i: (i, 0)),
            pl.BlockSpec((N_POI, EMBED), lambda i: (0, 0)),
        ],
        out_specs=[
            pl.BlockSpec((BM, EMBED), lambda i: (i, 0)),
            pl.BlockSpec((EMBED, N_POI), lambda i: (0, 0)),
            pl.BlockSpec((BM, PADW), lambda i: (i, 0)),
        ],
        out_shape=[
            jax.ShapeDtypeStruct((N_USER, EMBED), jnp.float32),
            jax.ShapeDtypeStruct((EMBED, N_POI), jnp.float32),
            jax.ShapeDtypeStruct((N_USER, PADW), jnp.bfloat16),
        ],
    )(A, x)


def _prop2_body(a_ref, x_ref, u_ref, pt_ref):
    i = pl.program_id(0)
    a = a_ref[...]
    u = jnp.dot(a, x_ref[...], preferred_element_type=jnp.float32)
    u_ref[...] = u
    pt = jax.lax.dot_general(u.astype(jnp.bfloat16), a,
                             (((0,), (0,)), ((), ())),
                             preferred_element_type=jnp.float32)

    @pl.when(i == 0)
    def _init():
        pt_ref[...] = pt

    @pl.when(i != 0)
    def _acc():
        pt_ref[...] = pt_ref[...] + pt


def _propagate2(A16, x16):
    return pl.pallas_call(
        _prop2_body,
        grid=(N_USER // BM2,),
        in_specs=[
            pl.BlockSpec((BM2, PADW), lambda i: (i, 0)),
            pl.BlockSpec((PADW, EMBED), lambda i: (0, 0)),
        ],
        out_specs=[
            pl.BlockSpec((BM2, EMBED), lambda i: (i, 0)),
            pl.BlockSpec((EMBED, PADW), lambda i: (0, 0)),
        ],
        out_shape=[
            jax.ShapeDtypeStruct((N_USER, EMBED), jnp.float32),
            jax.ShapeDtypeStruct((EMBED, PADW), jnp.float32),
        ],
    )(A16, x16)


def _transpose_body(xt_ref, x_ref):
    x_ref[...] = xt_ref[...].T


def _transpose(xt):
    n = xt.shape[1]
    return pl.pallas_call(
        _transpose_body,
        out_shape=jax.ShapeDtypeStruct((n, EMBED), jnp.float32),
    )(xt)


def _combine_user_body(e_ref, l1_ref, l2_ref, w_ref, out_ref):
    def norm(x):
        return x * jax.lax.rsqrt(jnp.sum(x * x, axis=1, keepdims=True))

    s = e_ref[...] + norm(l1_ref[...]) + norm(l2_ref[...])
    out_ref[...] = jnp.dot(s, w_ref[...].T, preferred_element_type=jnp.float32)


def _combine_user(embed, l1, l2, w):
    n = embed.shape[0]
    return pl.pallas_call(
        _combine_user_body,
        out_shape=jax.ShapeDtypeStruct((n, EMBED), jnp.float32),
    )(embed, l1, l2, w)


def _combine_poi_body(e_ref, l1t_ref, l2t_ref, w_ref, out_ref):
    def norm_t(xt):
        return xt * jax.lax.rsqrt(jnp.sum(xt * xt, axis=0, keepdims=True))

    st = e_ref[...].T + norm_t(l1t_ref[...]) + norm_t(l2t_ref[...])
    out_ref[...] = jnp.dot(w_ref[...], st, preferred_element_type=jnp.float32)


def _combine_poi_t(embed, l1t, l2t, w):
    n = embed.shape[0]
    return pl.pallas_call(
        _combine_poi_body,
        out_shape=jax.ShapeDtypeStruct((EMBED, n), jnp.float32),
    )(embed, l1t, l2t, w)


def _upw_body(bf_ref, pft_ref, out_ref):
    r = jnp.dot(bf_ref[...], pft_ref[...], preferred_element_type=jnp.float32)
    out_ref[...] = jnp.pad(r, ((0, 0), (0, PADW - N_POI)))


def _up_weight_padded(bf, pft):
    return pl.pallas_call(
        _upw_body,
        grid=(BATCH // BB,),
        in_specs=[
            pl.BlockSpec((BB, EMBED), lambda i: (i, 0)),
            pl.BlockSpec((EMBED, N_POI), lambda i: (0, 0)),
        ],
        out_specs=pl.BlockSpec((BB, PADW), lambda i: (i, 0)),
        out_shape=jax.ShapeDtypeStruct((BATCH, PADW), jnp.float32),
    )(bf, pft)


_NC, _NS = 2, 16
_NW = _NC * _NS
_BPW = BATCH // _NW


_GW = 128


def _sc_gather(table, idx):
    mesh = plsc.VectorSubcoreMesh(core_axis_name="c", subcore_axis_name="s")

    @functools.partial(
        pl.kernel, mesh=mesh,
        out_type=jax.ShapeDtypeStruct((BATCH, _GW), jnp.float32),
        scratch_types=[
            pltpu.VMEM((_BPW,), jnp.int32),
            pltpu.VMEM((_BPW, _GW), jnp.float32),
            pltpu.SemaphoreType.DMA,
        ],
    )
    def k(table_hbm, idx_hbm, out_hbm, idx_v, rows_v, sem):
        wid = lax.axis_index("s") * _NC + lax.axis_index("c")
        base = wid * _BPW
        pltpu.sync_copy(idx_hbm.at[pl.ds(base, _BPW)], idx_v)
        pltpu.async_copy(table_hbm.at[idx_v], rows_v, sem).wait()
        pltpu.sync_copy(rows_v, out_hbm.at[pl.ds(base, _BPW)])

    table_padded = jnp.pad(table, ((0, 0), (0, _GW - EMBED)))
    return k(table_padded, idx)[:, :EMBED]


def kernel(up_behavior_graph, user_embed, poi_embed, batch_user, W_user, W_poi):
    A = up_behavior_graph
    u1, p1t, a16 = _propagate1(A, poi_embed)
    p1 = _transpose(p1t)
    x16 = jnp.pad(p1.astype(jnp.bfloat16), ((0, PADW - N_POI), (0, 0)))
    u2, p2t_pad = _propagate2(a16, x16)
    p2t = p2t_pad[:, :N_POI]
    user_feature = _combine_user(user_embed, u1, u2, W_user)
    poi_feature_t = _combine_poi_t(poi_embed, p1t, p2t, W_poi)
    batch_user_feature = _sc_gather(user_feature, batch_user)
    up_weight = _up_weight_padded(batch_user_feature, poi_feature_t)[:, :N_POI]
    return (up_weight, user_feature)


# speedup vs baseline: 1.1059x; 1.1059x over previous
import functools

import jax
import jax.numpy as jnp
from jax import lax
from jax.experimental import pallas as pl
from jax.experimental.pallas import tpu as pltpu
from jax.experimental.pallas import tpu_sc as plsc

N_USER = 10000
N_POI = 10000
EMBED = 32
BATCH = 4096
BM = 400
BB = 512
PADW = 10112


def _prop_body(a_ref, x_ref, u_ref, pt_ref):
    i = pl.program_id(0)
    a = a_ref[...]
    u = jnp.dot(a, x_ref[...], preferred_element_type=jnp.float32)
    u_ref[...] = u
    pt = jax.lax.dot_general(u, a, (((0,), (0,)), ((), ())),
                             preferred_element_type=jnp.float32)

    @pl.when(i == 0)
    def _init():
        pt_ref[...] = pt

    @pl.when(i != 0)
    def _acc():
        pt_ref[...] = pt_ref[...] + pt


def _propagate(A, x):
    return pl.pallas_call(
        _prop_body,
        grid=(N_USER // BM,),
        in_specs=[
            pl.BlockSpec((BM, N_POI), lambda i: (i, 0)),
            pl.BlockSpec((N_POI, EMBED), lambda i: (0, 0)),
        ],
        out_specs=[
            pl.BlockSpec((BM, EMBED), lambda i: (i, 0)),
            pl.BlockSpec((EMBED, N_POI), lambda i: (0, 0)),
        ],
        out_shape=[
            jax.ShapeDtypeStruct((N_USER, EMBED), jnp.float32),
            jax.ShapeDtypeStruct((EMBED, N_POI), jnp.float32),
        ],
    )(A, x)


def _transpose_body(xt_ref, x_ref):
    x_ref[...] = xt_ref[...].T


def _transpose(xt):
    n = xt.shape[1]
    return pl.pallas_call(
        _transpose_body,
        out_shape=jax.ShapeDtypeStruct((n, EMBED), jnp.float32),
    )(xt)


def _combine_user_body(e_ref, l1_ref, l2_ref, w_ref, out_ref):
    def norm(x):
        return x * jax.lax.rsqrt(jnp.sum(x * x, axis=1, keepdims=True))

    s = e_ref[...] + norm(l1_ref[...]) + norm(l2_ref[...])
    out_ref[...] = jnp.dot(s, w_ref[...].T, preferred_element_type=jnp.float32)


def _combine_user(embed, l1, l2, w):
    n = embed.shape[0]
    return pl.pallas_call(
        _combine_user_body,
        out_shape=jax.ShapeDtypeStruct((n, EMBED), jnp.float32),
    )(embed, l1, l2, w)


def _combine_poi_body(e_ref, l1t_ref, l2t_ref, w_ref, out_ref):
    def norm_t(xt):
        return xt * jax.lax.rsqrt(jnp.sum(xt * xt, axis=0, keepdims=True))

    st = e_ref[...].T + norm_t(l1t_ref[...]) + norm_t(l2t_ref[...])
    out_ref[...] = jnp.dot(w_ref[...], st, preferred_element_type=jnp.float32)


def _combine_poi_t(embed, l1t, l2t, w):
    n = embed.shape[0]
    return pl.pallas_call(
        _combine_poi_body,
        out_shape=jax.ShapeDtypeStruct((EMBED, n), jnp.float32),
    )(embed, l1t, l2t, w)


def _upw_body(bf_ref, pft_ref, out_ref):
    r = jnp.dot(bf_ref[...], pft_ref[...], preferred_element_type=jnp.float32)
    out_ref[...] = jnp.pad(r, ((0, 0), (0, PADW - N_POI)))


def _up_weight_padded(bf, pft):
    return pl.pallas_call(
        _upw_body,
        grid=(BATCH // BB,),
        in_specs=[
            pl.BlockSpec((BB, EMBED), lambda i: (i, 0)),
            pl.BlockSpec((EMBED, N_POI), lambda i: (0, 0)),
        ],
        out_specs=pl.BlockSpec((BB, PADW), lambda i: (i, 0)),
        out_shape=jax.ShapeDtypeStruct((BATCH, PADW), jnp.float32),
    )(bf, pft)


_NC, _NS = 2, 16
_NW = _NC * _NS
_BPW = BATCH // _NW


_GW = 128


def _sc_gather(table, idx):
    mesh = plsc.VectorSubcoreMesh(core_axis_name="c", subcore_axis_name="s")

    @functools.partial(
        pl.kernel, mesh=mesh,
        out_type=jax.ShapeDtypeStruct((BATCH, _GW), jnp.float32),
        scratch_types=[
            pltpu.VMEM((_BPW,), jnp.int32),
            pltpu.VMEM((_BPW, _GW), jnp.float32),
            pltpu.SemaphoreType.DMA,
        ],
    )
    def k(table_hbm, idx_hbm, out_hbm, idx_v, rows_v, sem):
        wid = lax.axis_index("s") * _NC + lax.axis_index("c")
        base = wid * _BPW
        pltpu.sync_copy(idx_hbm.at[pl.ds(base, _BPW)], idx_v)
        pltpu.async_copy(table_hbm.at[idx_v], rows_v, sem).wait()
        pltpu.sync_copy(rows_v, out_hbm.at[pl.ds(base, _BPW)])

    table_padded = jnp.pad(table, ((0, 0), (0, _GW - EMBED)))
    return k(table_padded, idx)[:, :EMBED]


def kernel(up_behavior_graph, user_embed, poi_embed, batch_user, W_user, W_poi):
    A = up_behavior_graph
    u1, p1t = _propagate(A, poi_embed)
    p1 = _transpose(p1t)
    u2, p2t = _propagate(A, p1)
    user_feature = _combine_user(user_embed, u1, u2, W_user)
    poi_feature_t = _combine_poi_t(poi_embed, p1t, p2t, W_poi)
    batch_user_feature = _sc_gather(user_feature, batch_user)
    up_weight = _up_weight_padded(batch_user_feature, poi_feature_t)[:, :N_POI]
    return (up_weight, user_feature)
